# baseline (speedup 1.0000x reference)
import functools

import jax
import jax.numpy as jnp
from jax.experimental import pallas as pl
from jax.experimental.pallas import tpu as pltpu

BN_EPS = 1e-5


def _stats_kernel(x_ref, sx_ref, g_ref):
    t = pl.program_id(1)

    @pl.when(t == 0)
    def _init():
        sx_ref[...] = jnp.zeros_like(sx_ref)
        g_ref[...] = jnp.zeros_like(g_ref)

    x = x_ref[...]
    xf = x.astype(jnp.float32)
    sx_ref[...] += jnp.sum(xf, axis=(0, 2))[None, None, :]

    g_b = jax.lax.dot_general(
        x, x, (((2,), (2,)), ((0,), (0,))),
        preferred_element_type=jnp.float32)
    g_ref[...] += jnp.sum(g_b, axis=0)[None]


def _fold_kernel(sx_ref, g_ref, w_ref, gb_ref, aff_ref, *, inv_m):
    wf = w_ref[...]
    sx = sx_ref[0] + sx_ref[1]
    g = g_ref[0] + g_ref[1]
    mean_y = jnp.sum(wf * (sx * inv_m), axis=1, keepdims=True)
    wg = jnp.dot(wf, g, preferred_element_type=jnp.float32)
    e_y2 = jnp.sum(wg * wf, axis=1, keepdims=True) * inv_m
    var = e_y2 - mean_y * mean_y
    inv_std = jax.lax.rsqrt(var + BN_EPS)
    a = gb_ref[:, 0:1] * inv_std
    aff_ref[:, 0:1] = a
    aff_ref[:, 1:2] = gb_ref[:, 1:2] - mean_y * a


def _apply_kernel(x_ref, w_ref, aff_ref, o_ref):
    w = w_ref[...]
    scale = aff_ref[:, 0:1]
    shift = aff_ref[:, 1:2]
    for b in range(x_ref.shape[0]):
        y = jnp.dot(w, x_ref[b], preferred_element_type=jnp.float32)
        o_ref[b] = y * scale + shift


def _pick_bt(n):
    for bt in (8, 4, 2, 1):
        if n % (2 * bt) == 0:
            return bt, 2
    return 1, 1


def kernel(x_nchw, conv_weight, gamma, beta):
    stride = 2
    N, Cin, H, W = x_nchw.shape
    Cout = conv_weight.shape[0]

    xs = x_nchw[:, :, ::stride, ::stride]
    Ho, Wo = xs.shape[2], xs.shape[3]
    Hs = Ho * Wo
    x3 = xs.reshape(N, Cin, Hs).astype(jnp.bfloat16)

    w = conv_weight[:, :, 0, 0]
    w_bf = w.astype(jnp.bfloat16)
    gb = jnp.stack([gamma.astype(jnp.float32),
                    beta.astype(jnp.float32)], axis=1)

    BT, NC = _pick_bt(N)
    T2 = N // BT // NC
    inv_m = 1.0 / float(N * Hs)

    x_spec = pl.BlockSpec((BT, Cin, Hs), lambda c, t: (c * T2 + t, 0, 0))

    sx_part, g_part = pl.pallas_call(
        _stats_kernel,
        out_shape=(jax.ShapeDtypeStruct((NC, 1, Cin), jnp.float32),
                   jax.ShapeDtypeStruct((NC, Cin, Cin), jnp.float32)),
        grid=(NC, T2),
        in_specs=[x_spec],
        out_specs=(pl.BlockSpec((1, 1, Cin), lambda c, t: (c, 0, 0)),
                   pl.BlockSpec((1, Cin, Cin), lambda c, t: (c, 0, 0))),
        compiler_params=pltpu.CompilerParams(
            dimension_semantics=("parallel", "arbitrary")),
    )(x3)

    if NC == 1:
        sx_part = jnp.concatenate([sx_part, jnp.zeros_like(sx_part)], axis=0)
        g_part = jnp.concatenate([g_part, jnp.zeros_like(g_part)], axis=0)

    affine = pl.pallas_call(
        functools.partial(_fold_kernel, inv_m=inv_m),
        out_shape=jax.ShapeDtypeStruct((Cout, 2), jnp.float32),
    )(sx_part, g_part, w, gb)

    out3 = pl.pallas_call(
        _apply_kernel,
        out_shape=jax.ShapeDtypeStruct((N, Cout, Hs), x_nchw.dtype),
        grid=(NC, T2),
        in_specs=[x_spec,
                  pl.BlockSpec((Cout, Cin), lambda c, t: (0, 0)),
                  pl.BlockSpec((Cout, 2), lambda c, t: (0, 0))],
        out_specs=pl.BlockSpec((BT, Cout, Hs), lambda c, t: (c * T2 + t, 0, 0)),
        compiler_params=pltpu.CompilerParams(
            dimension_semantics=("parallel", "arbitrary")),
    )(x3, w_bf, affine)

    return out3.reshape(N, Cout, Ho, Wo)
